# baseline (speedup 1.0000x reference)
import functools

import jax
import jax.numpy as jnp
from jax import lax
from jax.experimental import pallas as pl
from jax.experimental.pallas import tpu as pltpu
from jax.experimental.pallas import tpu_sc as plsc

N_NODES = 10000
EMB = 128
N_EDGES = 320000
N_GRAPHS = 64
NUM_TASKS = 12
GAMMA = 0.4
ENV_W = 0.5

_NC = 2
_NS = 16
_NW = _NC * _NS
_CH = 128
_CPT = 80
_NE_PAD = _NW * _CPT * _CH
_BB = 20
_RPT = 624
_RTAIL = N_NODES - _NS * _RPT
_N_ACC = N_NODES + 32


def _agg_body(h_hbm, idx8_hbm, zeros_hbm, out0_hbm, out1_hbm,
              acc, rows0, qbuf, gs0):
    c = lax.axis_index("c")
    s = lax.axis_index("s")
    wid = s * _NC + c
    r0 = s * _RPT

    pltpu.sync_copy(zeros_hbm, acc.at[pl.ds(r0, _RPT)])

    @pl.when(s == _NS - 1)
    def _():
        pltpu.sync_copy(zeros_hbm.at[pl.ds(0, _RTAIL + 32)],
                        acc.at[pl.ds(_NS * _RPT, _RTAIL + 32)])

    plsc.subcore_barrier()

    qbase = wid * (_CPT // 4)

    def quad(q, carry):
        pltpu.sync_copy(idx8_hbm.at[qbase + q], qbuf)
        for i in range(4):
            pltpu.async_copy(h_hbm.at[qbuf.at[2 * i]], rows0, gs0).wait()
            pltpu.sync_copy(rows0, acc.at[qbuf.at[2 * i + 1]], add=True)
        return carry

    lax.fori_loop(0, _CPT // 4, quad, 0)

    plsc.subcore_barrier()

    @pl.when(c == 0)
    def _():
        pltpu.sync_copy(acc.at[pl.ds(r0, _RPT)], out0_hbm.at[pl.ds(r0, _RPT)])

        @pl.when(s == _NS - 1)
        def _():
            pltpu.sync_copy(acc.at[pl.ds(_NS * _RPT, _RTAIL)],
                            out0_hbm.at[pl.ds(_NS * _RPT, _RTAIL)])

    @pl.when(c == 1)
    def _():
        pltpu.sync_copy(acc.at[pl.ds(r0, _RPT)], out1_hbm.at[pl.ds(r0, _RPT)])

        @pl.when(s == _NS - 1)
        def _():
            pltpu.sync_copy(acc.at[pl.ds(_NS * _RPT, _RTAIL)],
                            out1_hbm.at[pl.ds(_NS * _RPT, _RTAIL)])


def _sc_aggregate(h, idx8, zeros_block):
    mesh = plsc.VectorSubcoreMesh(core_axis_name="c", subcore_axis_name="s")
    f = pl.kernel(
        _agg_body,
        out_type=(
            jax.ShapeDtypeStruct((N_NODES, EMB), jnp.float32),
            jax.ShapeDtypeStruct((N_NODES, EMB), jnp.float32),
        ),
        mesh=mesh,
        scratch_types=(
            [pltpu.VMEM_SHARED((_N_ACC, EMB), jnp.float32)]
            + [pltpu.VMEM((_CH, EMB), jnp.float32)]
            + [pltpu.VMEM((8, _CH), jnp.int32)]
            + [pltpu.SemaphoreType.DMA]
        ),
    )
    return f(h, idx8, zeros_block)


_BR = 400
_NB = N_NODES // _BR


def _gin_dense_body(eps_ref, h_ref, a0_ref, a1_ref, w1_ref, b1_ref,
                    w2_ref, b2_ref, o_ref, *, relu_out):
    z = (1.0 + eps_ref[0]) * h_ref[...] + a0_ref[...] + a1_ref[...]
    t = jnp.dot(z, w1_ref[...], preferred_element_type=jnp.float32) + b1_ref[...]
    t = jnp.maximum(t, 0.0)
    hn = jnp.dot(t, w2_ref[...], preferred_element_type=jnp.float32) + b2_ref[...]
    if relu_out:
        hn = jnp.maximum(hn, 0.0)
    o_ref[...] = hn + h_ref[...]


def _gin_dense(p, h, a0, a1, relu_out):
    body = functools.partial(_gin_dense_body, relu_out=relu_out)
    return pl.pallas_call(
        body,
        grid=(_NB,),
        in_specs=[
            pl.BlockSpec(memory_space=pltpu.SMEM),
            pl.BlockSpec((_BR, EMB), lambda i: (i, 0)),
            pl.BlockSpec((_BR, EMB), lambda i: (i, 0)),
            pl.BlockSpec((_BR, EMB), lambda i: (i, 0)),
            pl.BlockSpec((EMB, 2 * EMB), lambda i: (0, 0)),
            pl.BlockSpec((2 * EMB,), lambda i: (0,)),
            pl.BlockSpec((2 * EMB, EMB), lambda i: (0, 0)),
            pl.BlockSpec((EMB,), lambda i: (0,)),
        ],
        out_specs=pl.BlockSpec((_BR, EMB), lambda i: (i, 0)),
        out_shape=jax.ShapeDtypeStruct((N_NODES, EMB), jnp.float32),
    )(p["eps"].reshape(1), h, a0, a1,
      p["l1"]["w"], p["l1"]["b"], p["l2"]["w"], p["l2"]["b"])


def _pool_body(batch_ref, xr_ref, h_ref, gw1_ref, gb1_ref, gw2_ref, gb2_ref,
               hr_out, henv_out, loss_out,
               hr_acc, s_acc, gs_acc, cnt_acc):
    i = pl.program_id(0)

    @pl.when(i == 0)
    def _():
        hr_acc[...] = jnp.zeros_like(hr_acc)
        s_acc[...] = jnp.zeros_like(s_acc)
        gs_acc[...] = jnp.zeros_like(gs_acc)
        cnt_acc[...] = jnp.zeros_like(cnt_acc)

    b = batch_ref[0, 0, :]
    a = (lax.broadcasted_iota(jnp.int32, (N_GRAPHS, _BR), 0)
         == b[None, :]).astype(jnp.float32)

    xr = xr_ref[...]
    g1 = jnp.dot(xr, gw1_ref[...], preferred_element_type=jnp.float32) + gb1_ref[...]
    g1 = jnp.maximum(g1, 0.0)
    glog = jnp.dot(g1, gw2_ref[...], preferred_element_type=jnp.float32) + gb2_ref[...]
    gate = jax.nn.sigmoid(glog)

    ag = a * gate[:, 0][None, :]
    h = h_ref[...]
    hr_acc[...] += jnp.dot(ag, h, preferred_element_type=jnp.float32)
    s_acc[...] += jnp.dot(a, h, preferred_element_type=jnp.float32)
    gs_acc[...] += jnp.sum(ag, axis=1, keepdims=True)
    cnt_acc[...] += jnp.sum(a, axis=1, keepdims=True)

    @pl.when(i == _NB - 1)
    def _():
        hr = hr_acc[...]
        hr_out[...] = hr
        henv_out[...] = s_acc[...] - hr
        gs = gs_acc[...]
        r_num = gs + 1e-8
        e_num = (cnt_acc[...] - gs) + 1e-8
        ratio = r_num / (r_num + e_num)
        loss_out[...] = jnp.mean(jnp.abs(ratio - GAMMA)).reshape(1, 1)


def _pool(batch3, xr, h_node, gp):
    return pl.pallas_call(
        _pool_body,
        grid=(_NB,),
        in_specs=[
            pl.BlockSpec((1, 1, _BR), lambda i: (i, 0, 0)),
            pl.BlockSpec((_BR, EMB), lambda i: (i, 0)),
            pl.BlockSpec((_BR, EMB), lambda i: (i, 0)),
            pl.BlockSpec((EMB, 2 * EMB), lambda i: (0, 0)),
            pl.BlockSpec((2 * EMB,), lambda i: (0,)),
            pl.BlockSpec((2 * EMB, 1), lambda i: (0, 0)),
            pl.BlockSpec((1,), lambda i: (0,)),
        ],
        out_specs=[
            pl.BlockSpec((N_GRAPHS, EMB), lambda i: (0, 0)),
            pl.BlockSpec((N_GRAPHS, EMB), lambda i: (0, 0)),
            pl.BlockSpec((1, 1), lambda i: (0, 0)),
        ],
        out_shape=[
            jax.ShapeDtypeStruct((N_GRAPHS, EMB), jnp.float32),
            jax.ShapeDtypeStruct((N_GRAPHS, EMB), jnp.float32),
            jax.ShapeDtypeStruct((1, 1), jnp.float32),
        ],
        scratch_shapes=[
            pltpu.VMEM((N_GRAPHS, EMB), jnp.float32),
            pltpu.VMEM((N_GRAPHS, EMB), jnp.float32),
            pltpu.VMEM((N_GRAPHS, 1), jnp.float32),
            pltpu.VMEM((N_GRAPHS, 1), jnp.float32),
        ],
    )(batch3, xr, h_node, gp["l1"]["w"], gp["l1"]["b"],
      gp["l2"]["w"], gp["l2"]["b"])


def _pred_body(hr_ref, henv_ref, pw1_ref, pb1_ref, pw2_ref, pb2_ref,
               rep_out, rem_out):
    i = pl.program_id(0)
    henv = henv_ref[...]
    hr_row = hr_ref[pl.ds(i, 1), :]
    rep = hr_row + ENV_W * henv
    t = jnp.dot(rep, pw1_ref[...], preferred_element_type=jnp.float32) + pb1_ref[...]
    t = jnp.maximum(t, 0.0)
    rep_out[...] = jnp.dot(t, pw2_ref[...], preferred_element_type=jnp.float32) + pb2_ref[...]

    @pl.when(i == 0)
    def _():
        t2 = jnp.dot(hr_ref[...], pw1_ref[...], preferred_element_type=jnp.float32) + pb1_ref[...]
        t2 = jnp.maximum(t2, 0.0)
        rem_out[...] = jnp.dot(t2, pw2_ref[...], preferred_element_type=jnp.float32) + pb2_ref[...]


def _predict(hr, henv, pp):
    return pl.pallas_call(
        _pred_body,
        grid=(N_GRAPHS,),
        in_specs=[
            pl.BlockSpec((N_GRAPHS, EMB), lambda i: (0, 0)),
            pl.BlockSpec((N_GRAPHS, EMB), lambda i: (0, 0)),
            pl.BlockSpec((EMB, 2 * EMB), lambda i: (0, 0)),
            pl.BlockSpec((2 * EMB,), lambda i: (0,)),
            pl.BlockSpec((2 * EMB, NUM_TASKS), lambda i: (0, 0)),
            pl.BlockSpec((NUM_TASKS,), lambda i: (0,)),
        ],
        out_specs=[
            pl.BlockSpec((N_GRAPHS, NUM_TASKS), lambda i: (i, 0)),
            pl.BlockSpec((N_GRAPHS, NUM_TASKS), lambda i: (0, 0)),
        ],
        out_shape=[
            jax.ShapeDtypeStruct((N_GRAPHS * N_GRAPHS, NUM_TASKS), jnp.float32),
            jax.ShapeDtypeStruct((N_GRAPHS, NUM_TASKS), jnp.float32),
        ],
    )(hr, henv, pp["l1"]["w"], pp["l1"]["b"], pp["l2"]["w"], pp["l2"]["b"])


def kernel(x, edge_index, batch, params):
    src, dst = edge_index[0], edge_index[1]
    nchunk_real = N_EDGES // _CH
    npadc = _NW * _CPT - nchunk_real
    src2d = jnp.concatenate(
        [src.reshape(nchunk_real, _CH),
         jnp.zeros((npadc, _CH), jnp.int32)])
    pad_dst = (N_NODES
               + (jnp.arange(nchunk_real, nchunk_real + npadc,
                             dtype=jnp.int32) % _NW))[:, None]
    dst2d = jnp.concatenate(
        [dst.reshape(nchunk_real, _CH),
         jnp.broadcast_to(pad_dst, (npadc, _CH))])
    src_t = src2d.reshape(_CPT, _NW, _CH).transpose(1, 0, 2)
    dst_t = dst2d.reshape(_CPT, _NW, _CH).transpose(1, 0, 2)
    idx8 = jnp.stack([src_t, dst_t], axis=2).reshape(_NW * _CPT // 4, 8, _CH)
    zeros_block = jnp.zeros((_RPT, EMB), jnp.float32)
    batch3 = batch.reshape(_NB, 1, _BR)

    enc = params["enc"]
    rat = params["rat"]

    a0, a1 = _sc_aggregate(x, idx8, zeros_block)
    h = _gin_dense(enc[0], x, a0, a1, relu_out=True)
    xr = _gin_dense(rat[0], x, a0, a1, relu_out=True)

    for i in range(1, 5):
        a0, a1 = _sc_aggregate(h, idx8, zeros_block)
        h = _gin_dense(enc[i], h, a0, a1, relu_out=(i < 4))

    a0, a1 = _sc_aggregate(xr, idx8, zeros_block)
    xr = _gin_dense(rat[1], xr, a0, a1, relu_out=False)

    hr, henv, loss = _pool(batch3, xr, h, params["gate"])
    pred_rep, pred_rem = _predict(hr, henv, params["pred"])
    return pred_rep, pred_rem, loss.reshape(()), hr


# speedup vs baseline: 1.7982x; 1.7982x over previous
import functools

import jax
import jax.numpy as jnp
from jax import lax
from jax.experimental import pallas as pl
from jax.experimental.pallas import tpu as pltpu
from jax.experimental.pallas import tpu_sc as plsc

N_NODES = 10000
EMB = 128
N_EDGES = 320000
N_GRAPHS = 64
NUM_TASKS = 12
GAMMA = 0.4
ENV_W = 0.5

_NC = 2
_NS = 16
_NW = _NC * _NS
_CH = 128
_NCHUNK = N_EDGES // _CH
_NFULL = _NCHUNK // _NW
_NEXTRA = _NCHUNK - _NFULL * _NW
_RPT = 624
_RTAIL = N_NODES - _NS * _RPT
_N_ACC = N_NODES + 32


def _agg_body(h_hbm, src_hbm, dst_hbm, zeros_hbm, out0_hbm, out1_hbm,
              acc, sidx, didx, rows, sem):
    c = lax.axis_index("c")
    s = lax.axis_index("s")
    wid = s * _NC + c
    r0 = s * _RPT

    pltpu.sync_copy(zeros_hbm, acc.at[pl.ds(r0, _RPT)])

    @pl.when(s == _NS - 1)
    def _():
        pltpu.sync_copy(zeros_hbm.at[pl.ds(0, _RTAIL)],
                        acc.at[pl.ds(_NS * _RPT, _RTAIL)])

    plsc.subcore_barrier()

    def process(e0):
        pltpu.sync_copy(src_hbm.at[pl.ds(e0, _CH)], sidx)
        pltpu.sync_copy(dst_hbm.at[pl.ds(e0, _CH)], didx.at[0])
        pltpu.async_copy(h_hbm.at[sidx], rows, sem).wait()
        pltpu.sync_copy(rows, acc.at[didx.at[0]], add=True)

    def body(j, carry):
        process((j * _NW + wid) * _CH)
        return carry

    lax.fori_loop(0, _NFULL, body, 0)

    @pl.when(wid < _NEXTRA)
    def _():
        process((_NFULL * _NW + wid) * _CH)

    plsc.subcore_barrier()

    @pl.when(c == 0)
    def _():
        pltpu.sync_copy(acc.at[pl.ds(r0, _RPT)], out0_hbm.at[pl.ds(r0, _RPT)])

        @pl.when(s == _NS - 1)
        def _():
            pltpu.sync_copy(acc.at[pl.ds(_NS * _RPT, _RTAIL)],
                            out0_hbm.at[pl.ds(_NS * _RPT, _RTAIL)])

    @pl.when(c == 1)
    def _():
        pltpu.sync_copy(acc.at[pl.ds(r0, _RPT)], out1_hbm.at[pl.ds(r0, _RPT)])

        @pl.when(s == _NS - 1)
        def _():
            pltpu.sync_copy(acc.at[pl.ds(_NS * _RPT, _RTAIL)],
                            out1_hbm.at[pl.ds(_NS * _RPT, _RTAIL)])


def _sc_aggregate(h, src, dst, zeros_block):
    mesh = plsc.VectorSubcoreMesh(core_axis_name="c", subcore_axis_name="s")
    f = pl.kernel(
        _agg_body,
        out_type=(
            jax.ShapeDtypeStruct((N_NODES, EMB), jnp.float32),
            jax.ShapeDtypeStruct((N_NODES, EMB), jnp.float32),
        ),
        mesh=mesh,
        scratch_types=[
            pltpu.VMEM_SHARED((N_NODES, EMB), jnp.float32),
            pltpu.VMEM((_CH,), jnp.int32),
            pltpu.VMEM((1, _CH), jnp.int32),
            pltpu.VMEM((_CH, EMB), jnp.float32),
            pltpu.SemaphoreType.DMA,
        ],
    )
    return f(h, src, dst, zeros_block)


_BR = 400
_NB = N_NODES // _BR


def _gin_dense_body(eps_ref, h_ref, a0_ref, a1_ref, w1_ref, b1_ref,
                    w2_ref, b2_ref, o_ref, *, relu_out):
    z = (1.0 + eps_ref[0]) * h_ref[...] + a0_ref[...] + a1_ref[...]
    t = jnp.dot(z, w1_ref[...], preferred_element_type=jnp.float32) + b1_ref[...]
    t = jnp.maximum(t, 0.0)
    hn = jnp.dot(t, w2_ref[...], preferred_element_type=jnp.float32) + b2_ref[...]
    if relu_out:
        hn = jnp.maximum(hn, 0.0)
    o_ref[...] = hn + h_ref[...]


def _gin_dense(p, h, a0, a1, relu_out):
    body = functools.partial(_gin_dense_body, relu_out=relu_out)
    return pl.pallas_call(
        body,
        grid=(_NB,),
        in_specs=[
            pl.BlockSpec(memory_space=pltpu.SMEM),
            pl.BlockSpec((_BR, EMB), lambda i: (i, 0)),
            pl.BlockSpec((_BR, EMB), lambda i: (i, 0)),
            pl.BlockSpec((_BR, EMB), lambda i: (i, 0)),
            pl.BlockSpec((EMB, 2 * EMB), lambda i: (0, 0)),
            pl.BlockSpec((2 * EMB,), lambda i: (0,)),
            pl.BlockSpec((2 * EMB, EMB), lambda i: (0, 0)),
            pl.BlockSpec((EMB,), lambda i: (0,)),
        ],
        out_specs=pl.BlockSpec((_BR, EMB), lambda i: (i, 0)),
        out_shape=jax.ShapeDtypeStruct((N_NODES, EMB), jnp.float32),
    )(p["eps"].reshape(1), h, a0, a1,
      p["l1"]["w"], p["l1"]["b"], p["l2"]["w"], p["l2"]["b"])


def _pool_body(batch_ref, xr_ref, h_ref, gw1_ref, gb1_ref, gw2_ref, gb2_ref,
               hr_out, henv_out, loss_out,
               hr_acc, s_acc, gs_acc, cnt_acc):
    i = pl.program_id(0)

    @pl.when(i == 0)
    def _():
        hr_acc[...] = jnp.zeros_like(hr_acc)
        s_acc[...] = jnp.zeros_like(s_acc)
        gs_acc[...] = jnp.zeros_like(gs_acc)
        cnt_acc[...] = jnp.zeros_like(cnt_acc)

    b = batch_ref[0, 0, :]
    a = (lax.broadcasted_iota(jnp.int32, (N_GRAPHS, _BR), 0)
         == b[None, :]).astype(jnp.float32)

    xr = xr_ref[...]
    g1 = jnp.dot(xr, gw1_ref[...], preferred_element_type=jnp.float32) + gb1_ref[...]
    g1 = jnp.maximum(g1, 0.0)
    glog = jnp.dot(g1, gw2_ref[...], preferred_element_type=jnp.float32) + gb2_ref[...]
    gate = jax.nn.sigmoid(glog)

    ag = a * gate[:, 0][None, :]
    h = h_ref[...]
    hr_acc[...] += jnp.dot(ag, h, preferred_element_type=jnp.float32)
    s_acc[...] += jnp.dot(a, h, preferred_element_type=jnp.float32)
    gs_acc[...] += jnp.sum(ag, axis=1, keepdims=True)
    cnt_acc[...] += jnp.sum(a, axis=1, keepdims=True)

    @pl.when(i == _NB - 1)
    def _():
        hr = hr_acc[...]
        hr_out[...] = hr
        henv_out[...] = s_acc[...] - hr
        gs = gs_acc[...]
        r_num = gs + 1e-8
        e_num = (cnt_acc[...] - gs) + 1e-8
        ratio = r_num / (r_num + e_num)
        loss_out[...] = jnp.mean(jnp.abs(ratio - GAMMA)).reshape(1, 1)


def _pool(batch3, xr, h_node, gp):
    return pl.pallas_call(
        _pool_body,
        grid=(_NB,),
        in_specs=[
            pl.BlockSpec((1, 1, _BR), lambda i: (i, 0, 0)),
            pl.BlockSpec((_BR, EMB), lambda i: (i, 0)),
            pl.BlockSpec((_BR, EMB), lambda i: (i, 0)),
            pl.BlockSpec((EMB, 2 * EMB), lambda i: (0, 0)),
            pl.BlockSpec((2 * EMB,), lambda i: (0,)),
            pl.BlockSpec((2 * EMB, 1), lambda i: (0, 0)),
            pl.BlockSpec((1,), lambda i: (0,)),
        ],
        out_specs=[
            pl.BlockSpec((N_GRAPHS, EMB), lambda i: (0, 0)),
            pl.BlockSpec((N_GRAPHS, EMB), lambda i: (0, 0)),
            pl.BlockSpec((1, 1), lambda i: (0, 0)),
        ],
        out_shape=[
            jax.ShapeDtypeStruct((N_GRAPHS, EMB), jnp.float32),
            jax.ShapeDtypeStruct((N_GRAPHS, EMB), jnp.float32),
            jax.ShapeDtypeStruct((1, 1), jnp.float32),
        ],
        scratch_shapes=[
            pltpu.VMEM((N_GRAPHS, EMB), jnp.float32),
            pltpu.VMEM((N_GRAPHS, EMB), jnp.float32),
            pltpu.VMEM((N_GRAPHS, 1), jnp.float32),
            pltpu.VMEM((N_GRAPHS, 1), jnp.float32),
        ],
    )(batch3, xr, h_node, gp["l1"]["w"], gp["l1"]["b"],
      gp["l2"]["w"], gp["l2"]["b"])


def _pred_body(hr_ref, henv_ref, pw1_ref, pb1_ref, pw2_ref, pb2_ref,
               rep_out, rem_out):
    i = pl.program_id(0)
    henv = henv_ref[...]
    hr_row = hr_ref[pl.ds(i, 1), :]
    rep = hr_row + ENV_W * henv
    t = jnp.dot(rep, pw1_ref[...], preferred_element_type=jnp.float32) + pb1_ref[...]
    t = jnp.maximum(t, 0.0)
    rep_out[...] = jnp.dot(t, pw2_ref[...], preferred_element_type=jnp.float32) + pb2_ref[...]

    @pl.when(i == 0)
    def _():
        t2 = jnp.dot(hr_ref[...], pw1_ref[...], preferred_element_type=jnp.float32) + pb1_ref[...]
        t2 = jnp.maximum(t2, 0.0)
        rem_out[...] = jnp.dot(t2, pw2_ref[...], preferred_element_type=jnp.float32) + pb2_ref[...]


def _predict(hr, henv, pp):
    return pl.pallas_call(
        _pred_body,
        grid=(N_GRAPHS,),
        in_specs=[
            pl.BlockSpec((N_GRAPHS, EMB), lambda i: (0, 0)),
            pl.BlockSpec((N_GRAPHS, EMB), lambda i: (0, 0)),
            pl.BlockSpec((EMB, 2 * EMB), lambda i: (0, 0)),
            pl.BlockSpec((2 * EMB,), lambda i: (0,)),
            pl.BlockSpec((2 * EMB, NUM_TASKS), lambda i: (0, 0)),
            pl.BlockSpec((NUM_TASKS,), lambda i: (0,)),
        ],
        out_specs=[
            pl.BlockSpec((N_GRAPHS, NUM_TASKS), lambda i: (i, 0)),
            pl.BlockSpec((N_GRAPHS, NUM_TASKS), lambda i: (0, 0)),
        ],
        out_shape=[
            jax.ShapeDtypeStruct((N_GRAPHS * N_GRAPHS, NUM_TASKS), jnp.float32),
            jax.ShapeDtypeStruct((N_GRAPHS, NUM_TASKS), jnp.float32),
        ],
    )(hr, henv, pp["l1"]["w"], pp["l1"]["b"], pp["l2"]["w"], pp["l2"]["b"])


def kernel(x, edge_index, batch, params):
    src, dst = edge_index[0], edge_index[1]
    zeros_block = jnp.zeros((_RPT, EMB), jnp.float32)
    batch3 = batch.reshape(_NB, 1, _BR)

    enc = params["enc"]
    rat = params["rat"]

    a0, a1 = _sc_aggregate(x, src, dst, zeros_block)
    h = _gin_dense(enc[0], x, a0, a1, relu_out=True)
    xr = _gin_dense(rat[0], x, a0, a1, relu_out=True)

    for i in range(1, 5):
        a0, a1 = _sc_aggregate(h, src, dst, zeros_block)
        h = _gin_dense(enc[i], h, a0, a1, relu_out=(i < 4))

    a0, a1 = _sc_aggregate(xr, src, dst, zeros_block)
    xr = _gin_dense(rat[1], xr, a0, a1, relu_out=False)

    hr, henv, loss = _pool(batch3, xr, h, params["gate"])
    pred_rep, pred_rem = _predict(hr, henv, params["pred"])
    return pred_rep, pred_rem, loss.reshape(()), hr


# speedup vs baseline: 2.2828x; 1.2695x over previous
import functools

import jax
import jax.numpy as jnp
from jax import lax
from jax.experimental import pallas as pl
from jax.experimental.pallas import tpu as pltpu
from jax.experimental.pallas import tpu_sc as plsc

N_NODES = 10000
EMB = 128
N_EDGES = 320000
N_GRAPHS = 64
NUM_TASKS = 12
GAMMA = 0.4
ENV_W = 0.5

_NC = 2
_NS = 16
_NW = _NC * _NS
_CH = 128
_NCHUNK = N_EDGES // _CH
_NFULL = _NCHUNK // _NW
_NEXTRA = _NCHUNK - _NFULL * _NW
_RPT = 624
_RTAIL = N_NODES - _NS * _RPT
_N_ACC = N_NODES + 32


def _agg_body(h_hbm, src_hbm, dst_hbm, zeros_hbm, out0_hbm, out1_hbm,
              acc, sidx, didx, rows, sem, sidxb, didxb, rowsb, semb):
    c = lax.axis_index("c")
    s = lax.axis_index("s")
    wid = s * _NC + c
    r0 = s * _RPT

    pltpu.sync_copy(zeros_hbm, acc.at[pl.ds(r0, _RPT)])

    @pl.when(s == _NS - 1)
    def _():
        pltpu.sync_copy(zeros_hbm.at[pl.ds(0, _RTAIL)],
                        acc.at[pl.ds(_NS * _RPT, _RTAIL)])

    plsc.subcore_barrier()

    def process(e0):
        pltpu.sync_copy(src_hbm.at[pl.ds(e0, _CH)], sidx)
        pltpu.sync_copy(dst_hbm.at[pl.ds(e0, _CH)], didx.at[0])
        pltpu.async_copy(h_hbm.at[sidx], rows, sem).wait()
        pltpu.sync_copy(rows, acc.at[didx.at[0]], add=True)

    def body(k, carry):
        ea = ((2 * k) * _NW + wid) * _CH
        eb = ((2 * k + 1) * _NW + wid) * _CH
        pltpu.sync_copy(src_hbm.at[pl.ds(ea, _CH)], sidx)
        pltpu.sync_copy(dst_hbm.at[pl.ds(ea, _CH)], didx.at[0])
        ga = pltpu.async_copy(h_hbm.at[sidx], rows, sem)
        pltpu.sync_copy(src_hbm.at[pl.ds(eb, _CH)], sidxb)
        pltpu.sync_copy(dst_hbm.at[pl.ds(eb, _CH)], didxb.at[0])
        ga.wait()
        gb = pltpu.async_copy(h_hbm.at[sidxb], rowsb, semb)
        pltpu.sync_copy(rows, acc.at[didx.at[0]], add=True)
        gb.wait()
        pltpu.sync_copy(rowsb, acc.at[didxb.at[0]], add=True)
        return carry

    lax.fori_loop(0, _NFULL // 2, body, 0)

    @pl.when(wid < _NEXTRA)
    def _():
        process((_NFULL * _NW + wid) * _CH)

    plsc.subcore_barrier()

    @pl.when(c == 0)
    def _():
        pltpu.sync_copy(acc.at[pl.ds(r0, _RPT)], out0_hbm.at[pl.ds(r0, _RPT)])

        @pl.when(s == _NS - 1)
        def _():
            pltpu.sync_copy(acc.at[pl.ds(_NS * _RPT, _RTAIL)],
                            out0_hbm.at[pl.ds(_NS * _RPT, _RTAIL)])

    @pl.when(c == 1)
    def _():
        pltpu.sync_copy(acc.at[pl.ds(r0, _RPT)], out1_hbm.at[pl.ds(r0, _RPT)])

        @pl.when(s == _NS - 1)
        def _():
            pltpu.sync_copy(acc.at[pl.ds(_NS * _RPT, _RTAIL)],
                            out1_hbm.at[pl.ds(_NS * _RPT, _RTAIL)])


def _sc_aggregate(h, src, dst, zeros_block):
    mesh = plsc.VectorSubcoreMesh(core_axis_name="c", subcore_axis_name="s")
    f = pl.kernel(
        _agg_body,
        out_type=(
            jax.ShapeDtypeStruct((N_NODES, EMB), jnp.float32),
            jax.ShapeDtypeStruct((N_NODES, EMB), jnp.float32),
        ),
        mesh=mesh,
        scratch_types=[
            pltpu.VMEM_SHARED((N_NODES, EMB), jnp.float32),
            pltpu.VMEM((_CH,), jnp.int32),
            pltpu.VMEM((1, _CH), jnp.int32),
            pltpu.VMEM((_CH, EMB), jnp.float32),
            pltpu.SemaphoreType.DMA,
            pltpu.VMEM((_CH,), jnp.int32),
            pltpu.VMEM((1, _CH), jnp.int32),
            pltpu.VMEM((_CH, EMB), jnp.float32),
            pltpu.SemaphoreType.DMA,
        ],
    )
    return f(h, src, dst, zeros_block)


_BR = 400
_NB = N_NODES // _BR


def _gin_dense_body(eps_ref, h_ref, a0_ref, a1_ref, w1_ref, b1_ref,
                    w2_ref, b2_ref, o_ref, *, relu_out):
    z = (1.0 + eps_ref[0]) * h_ref[...] + a0_ref[...] + a1_ref[...]
    t = jnp.dot(z, w1_ref[...], preferred_element_type=jnp.float32) + b1_ref[...]
    t = jnp.maximum(t, 0.0)
    hn = jnp.dot(t, w2_ref[...], preferred_element_type=jnp.float32) + b2_ref[...]
    if relu_out:
        hn = jnp.maximum(hn, 0.0)
    o_ref[...] = hn + h_ref[...]


def _gin_dense(p, h, a0, a1, relu_out):
    body = functools.partial(_gin_dense_body, relu_out=relu_out)
    return pl.pallas_call(
        body,
        grid=(_NB,),
        in_specs=[
            pl.BlockSpec(memory_space=pltpu.SMEM),
            pl.BlockSpec((_BR, EMB), lambda i: (i, 0)),
            pl.BlockSpec((_BR, EMB), lambda i: (i, 0)),
            pl.BlockSpec((_BR, EMB), lambda i: (i, 0)),
            pl.BlockSpec((EMB, 2 * EMB), lambda i: (0, 0)),
            pl.BlockSpec((2 * EMB,), lambda i: (0,)),
            pl.BlockSpec((2 * EMB, EMB), lambda i: (0, 0)),
            pl.BlockSpec((EMB,), lambda i: (0,)),
        ],
        out_specs=pl.BlockSpec((_BR, EMB), lambda i: (i, 0)),
        out_shape=jax.ShapeDtypeStruct((N_NODES, EMB), jnp.float32),
    )(p["eps"].reshape(1), h, a0, a1,
      p["l1"]["w"], p["l1"]["b"], p["l2"]["w"], p["l2"]["b"])


def _pool_body(batch_ref, xr_ref, h_ref, gw1_ref, gb1_ref, gw2_ref, gb2_ref,
               hr_out, henv_out, loss_out,
               hr_acc, s_acc, gs_acc, cnt_acc):
    i = pl.program_id(0)

    @pl.when(i == 0)
    def _():
        hr_acc[...] = jnp.zeros_like(hr_acc)
        s_acc[...] = jnp.zeros_like(s_acc)
        gs_acc[...] = jnp.zeros_like(gs_acc)
        cnt_acc[...] = jnp.zeros_like(cnt_acc)

    b = batch_ref[0, 0, :]
    a = (lax.broadcasted_iota(jnp.int32, (N_GRAPHS, _BR), 0)
         == b[None, :]).astype(jnp.float32)

    xr = xr_ref[...]
    g1 = jnp.dot(xr, gw1_ref[...], preferred_element_type=jnp.float32) + gb1_ref[...]
    g1 = jnp.maximum(g1, 0.0)
    glog = jnp.dot(g1, gw2_ref[...], preferred_element_type=jnp.float32) + gb2_ref[...]
    gate = jax.nn.sigmoid(glog)

    ag = a * gate[:, 0][None, :]
    h = h_ref[...]
    hr_acc[...] += jnp.dot(ag, h, preferred_element_type=jnp.float32)
    s_acc[...] += jnp.dot(a, h, preferred_element_type=jnp.float32)
    gs_acc[...] += jnp.sum(ag, axis=1, keepdims=True)
    cnt_acc[...] += jnp.sum(a, axis=1, keepdims=True)

    @pl.when(i == _NB - 1)
    def _():
        hr = hr_acc[...]
        hr_out[...] = hr
        henv_out[...] = s_acc[...] - hr
        gs = gs_acc[...]
        r_num = gs + 1e-8
        e_num = (cnt_acc[...] - gs) + 1e-8
        ratio = r_num / (r_num + e_num)
        loss_out[...] = jnp.mean(jnp.abs(ratio - GAMMA)).reshape(1, 1)


def _pool(batch3, xr, h_node, gp):
    return pl.pallas_call(
        _pool_body,
        grid=(_NB,),
        in_specs=[
            pl.BlockSpec((1, 1, _BR), lambda i: (i, 0, 0)),
            pl.BlockSpec((_BR, EMB), lambda i: (i, 0)),
            pl.BlockSpec((_BR, EMB), lambda i: (i, 0)),
            pl.BlockSpec((EMB, 2 * EMB), lambda i: (0, 0)),
            pl.BlockSpec((2 * EMB,), lambda i: (0,)),
            pl.BlockSpec((2 * EMB, 1), lambda i: (0, 0)),
            pl.BlockSpec((1,), lambda i: (0,)),
        ],
        out_specs=[
            pl.BlockSpec((N_GRAPHS, EMB), lambda i: (0, 0)),
            pl.BlockSpec((N_GRAPHS, EMB), lambda i: (0, 0)),
            pl.BlockSpec((1, 1), lambda i: (0, 0)),
        ],
        out_shape=[
            jax.ShapeDtypeStruct((N_GRAPHS, EMB), jnp.float32),
            jax.ShapeDtypeStruct((N_GRAPHS, EMB), jnp.float32),
            jax.ShapeDtypeStruct((1, 1), jnp.float32),
        ],
        scratch_shapes=[
            pltpu.VMEM((N_GRAPHS, EMB), jnp.float32),
            pltpu.VMEM((N_GRAPHS, EMB), jnp.float32),
            pltpu.VMEM((N_GRAPHS, 1), jnp.float32),
            pltpu.VMEM((N_GRAPHS, 1), jnp.float32),
        ],
    )(batch3, xr, h_node, gp["l1"]["w"], gp["l1"]["b"],
      gp["l2"]["w"], gp["l2"]["b"])


def _pred_body(hr_ref, henv_ref, pw1_ref, pb1_ref, pw2_ref, pb2_ref,
               rep_out, rem_out):
    i = pl.program_id(0)
    henv = henv_ref[...]
    hr_row = hr_ref[pl.ds(i, 1), :]
    rep = hr_row + ENV_W * henv
    t = jnp.dot(rep, pw1_ref[...], preferred_element_type=jnp.float32) + pb1_ref[...]
    t = jnp.maximum(t, 0.0)
    rep_out[...] = jnp.dot(t, pw2_ref[...], preferred_element_type=jnp.float32) + pb2_ref[...]

    @pl.when(i == 0)
    def _():
        t2 = jnp.dot(hr_ref[...], pw1_ref[...], preferred_element_type=jnp.float32) + pb1_ref[...]
        t2 = jnp.maximum(t2, 0.0)
        rem_out[...] = jnp.dot(t2, pw2_ref[...], preferred_element_type=jnp.float32) + pb2_ref[...]


def _predict(hr, henv, pp):
    return pl.pallas_call(
        _pred_body,
        grid=(N_GRAPHS,),
        in_specs=[
            pl.BlockSpec((N_GRAPHS, EMB), lambda i: (0, 0)),
            pl.BlockSpec((N_GRAPHS, EMB), lambda i: (0, 0)),
            pl.BlockSpec((EMB, 2 * EMB), lambda i: (0, 0)),
            pl.BlockSpec((2 * EMB,), lambda i: (0,)),
            pl.BlockSpec((2 * EMB, NUM_TASKS), lambda i: (0, 0)),
            pl.BlockSpec((NUM_TASKS,), lambda i: (0,)),
        ],
        out_specs=[
            pl.BlockSpec((N_GRAPHS, NUM_TASKS), lambda i: (i, 0)),
            pl.BlockSpec((N_GRAPHS, NUM_TASKS), lambda i: (0, 0)),
        ],
        out_shape=[
            jax.ShapeDtypeStruct((N_GRAPHS * N_GRAPHS, NUM_TASKS), jnp.float32),
            jax.ShapeDtypeStruct((N_GRAPHS, NUM_TASKS), jnp.float32),
        ],
    )(hr, henv, pp["l1"]["w"], pp["l1"]["b"], pp["l2"]["w"], pp["l2"]["b"])


def kernel(x, edge_index, batch, params):
    src, dst = edge_index[0], edge_index[1]
    zeros_block = jnp.zeros((_RPT, EMB), jnp.float32)
    batch3 = batch.reshape(_NB, 1, _BR)

    enc = params["enc"]
    rat = params["rat"]

    a0, a1 = _sc_aggregate(x, src, dst, zeros_block)
    h = _gin_dense(enc[0], x, a0, a1, relu_out=True)
    xr = _gin_dense(rat[0], x, a0, a1, relu_out=True)

    for i in range(1, 5):
        a0, a1 = _sc_aggregate(h, src, dst, zeros_block)
        h = _gin_dense(enc[i], h, a0, a1, relu_out=(i < 4))

    a0, a1 = _sc_aggregate(xr, src, dst, zeros_block)
    xr = _gin_dense(rat[1], xr, a0, a1, relu_out=False)

    hr, henv, loss = _pool(batch3, xr, h, params["gate"])
    pred_rep, pred_rem = _predict(hr, henv, params["pred"])
    return pred_rep, pred_rem, loss.reshape(()), hr


# speedup vs baseline: 2.4253x; 1.0624x over previous
import functools

import jax
import jax.numpy as jnp
from jax import lax
from jax.experimental import pallas as pl
from jax.experimental.pallas import tpu as pltpu
from jax.experimental.pallas import tpu_sc as plsc

N_NODES = 10000
EMB = 128
N_EDGES = 320000
N_GRAPHS = 64
NUM_TASKS = 12
GAMMA = 0.4
ENV_W = 0.5

_NC = 2
_NS = 16
_NW = _NC * _NS
_CH = 128
_NCHUNK = N_EDGES // _CH
_NFULL = _NCHUNK // _NW
_NEXTRA = _NCHUNK - _NFULL * _NW
_RPT = 624
_RTAIL = N_NODES - _NS * _RPT
_N_ACC = N_NODES + 32


def _agg_body(h_hbm, src_hbm, dst_hbm, zeros_hbm, out0_hbm, out1_hbm,
              acc, sidx, didx, rows, sem, sidxb, didxb, rowsb, semb,
              sidxc, didxc, rowsc, semc):
    c = lax.axis_index("c")
    s = lax.axis_index("s")
    wid = s * _NC + c
    r0 = s * _RPT

    pltpu.sync_copy(zeros_hbm, acc.at[pl.ds(r0, _RPT)])

    @pl.when(s == _NS - 1)
    def _():
        pltpu.sync_copy(zeros_hbm.at[pl.ds(0, _RTAIL)],
                        acc.at[pl.ds(_NS * _RPT, _RTAIL)])

    plsc.subcore_barrier()

    def process(e0):
        pltpu.sync_copy(src_hbm.at[pl.ds(e0, _CH)], sidx)
        pltpu.sync_copy(dst_hbm.at[pl.ds(e0, _CH)], didx.at[0])
        pltpu.async_copy(h_hbm.at[sidx], rows, sem).wait()
        pltpu.sync_copy(rows, acc.at[didx.at[0]], add=True)

    def body(k, carry):
        ea = ((3 * k) * _NW + wid) * _CH
        eb = ((3 * k + 1) * _NW + wid) * _CH
        ec = ((3 * k + 2) * _NW + wid) * _CH
        pltpu.sync_copy(src_hbm.at[pl.ds(ea, _CH)], sidx)
        pltpu.sync_copy(dst_hbm.at[pl.ds(ea, _CH)], didx.at[0])
        ga = pltpu.async_copy(h_hbm.at[sidx], rows, sem)
        pltpu.sync_copy(src_hbm.at[pl.ds(eb, _CH)], sidxb)
        pltpu.sync_copy(dst_hbm.at[pl.ds(eb, _CH)], didxb.at[0])
        ga.wait()
        gb = pltpu.async_copy(h_hbm.at[sidxb], rowsb, semb)
        pltpu.sync_copy(rows, acc.at[didx.at[0]], add=True)
        pltpu.sync_copy(src_hbm.at[pl.ds(ec, _CH)], sidxc)
        pltpu.sync_copy(dst_hbm.at[pl.ds(ec, _CH)], didxc.at[0])
        gb.wait()
        gc = pltpu.async_copy(h_hbm.at[sidxc], rowsc, semc)
        pltpu.sync_copy(rowsb, acc.at[didxb.at[0]], add=True)
        gc.wait()
        pltpu.sync_copy(rowsc, acc.at[didxc.at[0]], add=True)
        return carry

    lax.fori_loop(0, _NFULL // 3, body, 0)

    @pl.when(wid < _NEXTRA)
    def _():
        process((_NFULL * _NW + wid) * _CH)

    plsc.subcore_barrier()

    @pl.when(c == 0)
    def _():
        pltpu.sync_copy(acc.at[pl.ds(r0, _RPT)], out0_hbm.at[pl.ds(r0, _RPT)])

        @pl.when(s == _NS - 1)
        def _():
            pltpu.sync_copy(acc.at[pl.ds(_NS * _RPT, _RTAIL)],
                            out0_hbm.at[pl.ds(_NS * _RPT, _RTAIL)])

    @pl.when(c == 1)
    def _():
        pltpu.sync_copy(acc.at[pl.ds(r0, _RPT)], out1_hbm.at[pl.ds(r0, _RPT)])

        @pl.when(s == _NS - 1)
        def _():
            pltpu.sync_copy(acc.at[pl.ds(_NS * _RPT, _RTAIL)],
                            out1_hbm.at[pl.ds(_NS * _RPT, _RTAIL)])


def _sc_aggregate(h, src, dst, zeros_block):
    mesh = plsc.VectorSubcoreMesh(core_axis_name="c", subcore_axis_name="s")
    f = pl.kernel(
        _agg_body,
        out_type=(
            jax.ShapeDtypeStruct((N_NODES, EMB), jnp.float32),
            jax.ShapeDtypeStruct((N_NODES, EMB), jnp.float32),
        ),
        mesh=mesh,
        scratch_types=[
            pltpu.VMEM_SHARED((N_NODES, EMB), jnp.float32),
            pltpu.VMEM((_CH,), jnp.int32),
            pltpu.VMEM((1, _CH), jnp.int32),
            pltpu.VMEM((_CH, EMB), jnp.float32),
            pltpu.SemaphoreType.DMA,
            pltpu.VMEM((_CH,), jnp.int32),
            pltpu.VMEM((1, _CH), jnp.int32),
            pltpu.VMEM((_CH, EMB), jnp.float32),
            pltpu.SemaphoreType.DMA,
            pltpu.VMEM((_CH,), jnp.int32),
            pltpu.VMEM((1, _CH), jnp.int32),
            pltpu.VMEM((_CH, EMB), jnp.float32),
            pltpu.SemaphoreType.DMA,
        ],
    )
    return f(h, src, dst, zeros_block)


_BR = 400
_NB = N_NODES // _BR


def _gin_dense_body(eps_ref, h_ref, a0_ref, a1_ref, w1_ref, b1_ref,
                    w2_ref, b2_ref, o_ref, *, relu_out):
    z = (1.0 + eps_ref[0]) * h_ref[...] + a0_ref[...] + a1_ref[...]
    t = jnp.dot(z, w1_ref[...], preferred_element_type=jnp.float32) + b1_ref[...]
    t = jnp.maximum(t, 0.0)
    hn = jnp.dot(t, w2_ref[...], preferred_element_type=jnp.float32) + b2_ref[...]
    if relu_out:
        hn = jnp.maximum(hn, 0.0)
    o_ref[...] = hn + h_ref[...]


def _gin_dense(p, h, a0, a1, relu_out):
    body = functools.partial(_gin_dense_body, relu_out=relu_out)
    return pl.pallas_call(
        body,
        grid=(_NB,),
        in_specs=[
            pl.BlockSpec(memory_space=pltpu.SMEM),
            pl.BlockSpec((_BR, EMB), lambda i: (i, 0)),
            pl.BlockSpec((_BR, EMB), lambda i: (i, 0)),
            pl.BlockSpec((_BR, EMB), lambda i: (i, 0)),
            pl.BlockSpec((EMB, 2 * EMB), lambda i: (0, 0)),
            pl.BlockSpec((2 * EMB,), lambda i: (0,)),
            pl.BlockSpec((2 * EMB, EMB), lambda i: (0, 0)),
            pl.BlockSpec((EMB,), lambda i: (0,)),
        ],
        out_specs=pl.BlockSpec((_BR, EMB), lambda i: (i, 0)),
        out_shape=jax.ShapeDtypeStruct((N_NODES, EMB), jnp.float32),
    )(p["eps"].reshape(1), h, a0, a1,
      p["l1"]["w"], p["l1"]["b"], p["l2"]["w"], p["l2"]["b"])


def _pool_body(batch_ref, xr_ref, h_ref, gw1_ref, gb1_ref, gw2_ref, gb2_ref,
               hr_out, henv_out, loss_out,
               hr_acc, s_acc, gs_acc, cnt_acc):
    i = pl.program_id(0)

    @pl.when(i == 0)
    def _():
        hr_acc[...] = jnp.zeros_like(hr_acc)
        s_acc[...] = jnp.zeros_like(s_acc)
        gs_acc[...] = jnp.zeros_like(gs_acc)
        cnt_acc[...] = jnp.zeros_like(cnt_acc)

    b = batch_ref[0, 0, :]
    a = (lax.broadcasted_iota(jnp.int32, (N_GRAPHS, _BR), 0)
         == b[None, :]).astype(jnp.float32)

    xr = xr_ref[...]
    g1 = jnp.dot(xr, gw1_ref[...], preferred_element_type=jnp.float32) + gb1_ref[...]
    g1 = jnp.maximum(g1, 0.0)
    glog = jnp.dot(g1, gw2_ref[...], preferred_element_type=jnp.float32) + gb2_ref[...]
    gate = jax.nn.sigmoid(glog)

    ag = a * gate[:, 0][None, :]
    h = h_ref[...]
    hr_acc[...] += jnp.dot(ag, h, preferred_element_type=jnp.float32)
    s_acc[...] += jnp.dot(a, h, preferred_element_type=jnp.float32)
    gs_acc[...] += jnp.sum(ag, axis=1, keepdims=True)
    cnt_acc[...] += jnp.sum(a, axis=1, keepdims=True)

    @pl.when(i == _NB - 1)
    def _():
        hr = hr_acc[...]
        hr_out[...] = hr
        henv_out[...] = s_acc[...] - hr
        gs = gs_acc[...]
        r_num = gs + 1e-8
        e_num = (cnt_acc[...] - gs) + 1e-8
        ratio = r_num / (r_num + e_num)
        loss_out[...] = jnp.mean(jnp.abs(ratio - GAMMA)).reshape(1, 1)


def _pool(batch3, xr, h_node, gp):
    return pl.pallas_call(
        _pool_body,
        grid=(_NB,),
        in_specs=[
            pl.BlockSpec((1, 1, _BR), lambda i: (i, 0, 0)),
            pl.BlockSpec((_BR, EMB), lambda i: (i, 0)),
            pl.BlockSpec((_BR, EMB), lambda i: (i, 0)),
            pl.BlockSpec((EMB, 2 * EMB), lambda i: (0, 0)),
            pl.BlockSpec((2 * EMB,), lambda i: (0,)),
            pl.BlockSpec((2 * EMB, 1), lambda i: (0, 0)),
            pl.BlockSpec((1,), lambda i: (0,)),
        ],
        out_specs=[
            pl.BlockSpec((N_GRAPHS, EMB), lambda i: (0, 0)),
            pl.BlockSpec((N_GRAPHS, EMB), lambda i: (0, 0)),
            pl.BlockSpec((1, 1), lambda i: (0, 0)),
        ],
        out_shape=[
            jax.ShapeDtypeStruct((N_GRAPHS, EMB), jnp.float32),
            jax.ShapeDtypeStruct((N_GRAPHS, EMB), jnp.float32),
            jax.ShapeDtypeStruct((1, 1), jnp.float32),
        ],
        scratch_shapes=[
            pltpu.VMEM((N_GRAPHS, EMB), jnp.float32),
            pltpu.VMEM((N_GRAPHS, EMB), jnp.float32),
            pltpu.VMEM((N_GRAPHS, 1), jnp.float32),
            pltpu.VMEM((N_GRAPHS, 1), jnp.float32),
        ],
    )(batch3, xr, h_node, gp["l1"]["w"], gp["l1"]["b"],
      gp["l2"]["w"], gp["l2"]["b"])


def _pred_body(hr_ref, henv_ref, pw1_ref, pb1_ref, pw2_ref, pb2_ref,
               rep_out, rem_out):
    i = pl.program_id(0)
    henv = henv_ref[...]
    hr_row = hr_ref[pl.ds(i, 1), :]
    rep = hr_row + ENV_W * henv
    t = jnp.dot(rep, pw1_ref[...], preferred_element_type=jnp.float32) + pb1_ref[...]
    t = jnp.maximum(t, 0.0)
    rep_out[...] = jnp.dot(t, pw2_ref[...], preferred_element_type=jnp.float32) + pb2_ref[...]

    @pl.when(i == 0)
    def _():
        t2 = jnp.dot(hr_ref[...], pw1_ref[...], preferred_element_type=jnp.float32) + pb1_ref[...]
        t2 = jnp.maximum(t2, 0.0)
        rem_out[...] = jnp.dot(t2, pw2_ref[...], preferred_element_type=jnp.float32) + pb2_ref[...]


def _predict(hr, henv, pp):
    return pl.pallas_call(
        _pred_body,
        grid=(N_GRAPHS,),
        in_specs=[
            pl.BlockSpec((N_GRAPHS, EMB), lambda i: (0, 0)),
            pl.BlockSpec((N_GRAPHS, EMB), lambda i: (0, 0)),
            pl.BlockSpec((EMB, 2 * EMB), lambda i: (0, 0)),
            pl.BlockSpec((2 * EMB,), lambda i: (0,)),
            pl.BlockSpec((2 * EMB, NUM_TASKS), lambda i: (0, 0)),
            pl.BlockSpec((NUM_TASKS,), lambda i: (0,)),
        ],
        out_specs=[
            pl.BlockSpec((N_GRAPHS, NUM_TASKS), lambda i: (i, 0)),
            pl.BlockSpec((N_GRAPHS, NUM_TASKS), lambda i: (0, 0)),
        ],
        out_shape=[
            jax.ShapeDtypeStruct((N_GRAPHS * N_GRAPHS, NUM_TASKS), jnp.float32),
            jax.ShapeDtypeStruct((N_GRAPHS, NUM_TASKS), jnp.float32),
        ],
    )(hr, henv, pp["l1"]["w"], pp["l1"]["b"], pp["l2"]["w"], pp["l2"]["b"])


def kernel(x, edge_index, batch, params):
    src, dst = edge_index[0], edge_index[1]
    zeros_block = jnp.zeros((_RPT, EMB), jnp.float32)
    batch3 = batch.reshape(_NB, 1, _BR)

    enc = params["enc"]
    rat = params["rat"]

    a0, a1 = _sc_aggregate(x, src, dst, zeros_block)
    h = _gin_dense(enc[0], x, a0, a1, relu_out=True)
    xr = _gin_dense(rat[0], x, a0, a1, relu_out=True)

    for i in range(1, 5):
        a0, a1 = _sc_aggregate(h, src, dst, zeros_block)
        h = _gin_dense(enc[i], h, a0, a1, relu_out=(i < 4))

    a0, a1 = _sc_aggregate(xr, src, dst, zeros_block)
    xr = _gin_dense(rat[1], xr, a0, a1, relu_out=False)

    hr, henv, loss = _pool(batch3, xr, h, params["gate"])
    pred_rep, pred_rem = _predict(hr, henv, params["pred"])
    return pred_rep, pred_rem, loss.reshape(()), hr


# speedup vs baseline: 2.6238x; 1.0819x over previous
import functools

import jax
import jax.numpy as jnp
from jax import lax
from jax.experimental import pallas as pl
from jax.experimental.pallas import tpu as pltpu
from jax.experimental.pallas import tpu_sc as plsc

N_NODES = 10000
EMB = 128
N_EDGES = 320000
N_GRAPHS = 64
NUM_TASKS = 12
GAMMA = 0.4
ENV_W = 0.5

_NC = 2
_NS = 16
_NW = _NC * _NS
_CH = 128
_NCHUNK = N_EDGES // _CH
_NFULL = _NCHUNK // _NW
_NEXTRA = _NCHUNK - _NFULL * _NW
_RPT = 624
_RTAIL = N_NODES - _NS * _RPT
_N_ACC = N_NODES + 32


def _agg_body(h_hbm, src_hbm, dst_hbm, zeros_hbm, out0_hbm, out1_hbm,
              acc, sidx, didx, rows, sem, sidxb, didxb, rowsb, semb,
              sidxc, didxc, rowsc, semc):
    c = lax.axis_index("c")
    s = lax.axis_index("s")
    wid = s * _NC + c
    r0 = s * _RPT

    pltpu.sync_copy(zeros_hbm, acc.at[pl.ds(r0, _RPT)])

    @pl.when(s == _NS - 1)
    def _():
        pltpu.sync_copy(zeros_hbm.at[pl.ds(0, _RTAIL)],
                        acc.at[pl.ds(_NS * _RPT, _RTAIL)])

    plsc.subcore_barrier()

    def process(e0):
        pltpu.sync_copy(src_hbm.at[pl.ds(e0, _CH)], sidx)
        pltpu.sync_copy(dst_hbm.at[pl.ds(e0, _CH)], didx.at[0])
        pltpu.async_copy(h_hbm.at[sidx], rows, sem).wait()
        pltpu.sync_copy(rows, acc.at[didx.at[0]], add=True)

    pltpu.sync_copy(src_hbm.at[pl.ds(wid * _CH, _CH)], sidx)
    pltpu.sync_copy(dst_hbm.at[pl.ds(wid * _CH, _CH)], didx.at[0])

    def body(k, carry):
        eb = ((3 * k + 1) * _NW + wid) * _CH
        ec = ((3 * k + 2) * _NW + wid) * _CH
        ea_next = (jnp.minimum((3 * k + 3) * _NW + wid, _NCHUNK - 1)) * _CH
        ga = pltpu.async_copy(h_hbm.at[sidx], rows, sem)
        pltpu.sync_copy(src_hbm.at[pl.ds(eb, _CH)], sidxb)
        pltpu.sync_copy(dst_hbm.at[pl.ds(eb, _CH)], didxb.at[0])
        ga.wait()
        gb = pltpu.async_copy(h_hbm.at[sidxb], rowsb, semb)
        pltpu.sync_copy(rows, acc.at[didx.at[0]], add=True)
        pltpu.sync_copy(src_hbm.at[pl.ds(ec, _CH)], sidxc)
        pltpu.sync_copy(dst_hbm.at[pl.ds(ec, _CH)], didxc.at[0])
        gb.wait()
        gc = pltpu.async_copy(h_hbm.at[sidxc], rowsc, semc)
        pltpu.sync_copy(rowsb, acc.at[didxb.at[0]], add=True)
        pltpu.sync_copy(src_hbm.at[pl.ds(ea_next, _CH)], sidx)
        pltpu.sync_copy(dst_hbm.at[pl.ds(ea_next, _CH)], didx.at[0])
        gc.wait()
        pltpu.sync_copy(rowsc, acc.at[didxc.at[0]], add=True)
        return carry

    lax.fori_loop(0, _NFULL // 3, body, 0)

    @pl.when(wid < _NEXTRA)
    def _():
        process((_NFULL * _NW + wid) * _CH)

    plsc.subcore_barrier()

    @pl.when(c == 0)
    def _():
        pltpu.sync_copy(acc.at[pl.ds(r0, _RPT)], out0_hbm.at[pl.ds(r0, _RPT)])

        @pl.when(s == _NS - 1)
        def _():
            pltpu.sync_copy(acc.at[pl.ds(_NS * _RPT, _RTAIL)],
                            out0_hbm.at[pl.ds(_NS * _RPT, _RTAIL)])

    @pl.when(c == 1)
    def _():
        pltpu.sync_copy(acc.at[pl.ds(r0, _RPT)], out1_hbm.at[pl.ds(r0, _RPT)])

        @pl.when(s == _NS - 1)
        def _():
            pltpu.sync_copy(acc.at[pl.ds(_NS * _RPT, _RTAIL)],
                            out1_hbm.at[pl.ds(_NS * _RPT, _RTAIL)])


def _sc_aggregate(h, src, dst, zeros_block):
    mesh = plsc.VectorSubcoreMesh(core_axis_name="c", subcore_axis_name="s")
    f = pl.kernel(
        _agg_body,
        out_type=(
            jax.ShapeDtypeStruct((N_NODES, EMB), jnp.float32),
            jax.ShapeDtypeStruct((N_NODES, EMB), jnp.float32),
        ),
        mesh=mesh,
        scratch_types=[
            pltpu.VMEM_SHARED((N_NODES, EMB), jnp.float32),
            pltpu.VMEM((_CH,), jnp.int32),
            pltpu.VMEM((1, _CH), jnp.int32),
            pltpu.VMEM((_CH, EMB), jnp.float32),
            pltpu.SemaphoreType.DMA,
            pltpu.VMEM((_CH,), jnp.int32),
            pltpu.VMEM((1, _CH), jnp.int32),
            pltpu.VMEM((_CH, EMB), jnp.float32),
            pltpu.SemaphoreType.DMA,
            pltpu.VMEM((_CH,), jnp.int32),
            pltpu.VMEM((1, _CH), jnp.int32),
            pltpu.VMEM((_CH, EMB), jnp.float32),
            pltpu.SemaphoreType.DMA,
        ],
    )
    return f(h, src, dst, zeros_block)


_BR = 400
_NB = N_NODES // _BR


def _gin_dense_body(eps_ref, h_ref, a0_ref, a1_ref, w1_ref, b1_ref,
                    w2_ref, b2_ref, o_ref, *, relu_out):
    z = (1.0 + eps_ref[0]) * h_ref[...] + a0_ref[...] + a1_ref[...]
    t = jnp.dot(z, w1_ref[...], preferred_element_type=jnp.float32) + b1_ref[...]
    t = jnp.maximum(t, 0.0)
    hn = jnp.dot(t, w2_ref[...], preferred_element_type=jnp.float32) + b2_ref[...]
    if relu_out:
        hn = jnp.maximum(hn, 0.0)
    o_ref[...] = hn + h_ref[...]


def _gin_dense(p, h, a0, a1, relu_out):
    body = functools.partial(_gin_dense_body, relu_out=relu_out)
    return pl.pallas_call(
        body,
        grid=(_NB,),
        in_specs=[
            pl.BlockSpec(memory_space=pltpu.SMEM),
            pl.BlockSpec((_BR, EMB), lambda i: (i, 0)),
            pl.BlockSpec((_BR, EMB), lambda i: (i, 0)),
            pl.BlockSpec((_BR, EMB), lambda i: (i, 0)),
            pl.BlockSpec((EMB, 2 * EMB), lambda i: (0, 0)),
            pl.BlockSpec((2 * EMB,), lambda i: (0,)),
            pl.BlockSpec((2 * EMB, EMB), lambda i: (0, 0)),
            pl.BlockSpec((EMB,), lambda i: (0,)),
        ],
        out_specs=pl.BlockSpec((_BR, EMB), lambda i: (i, 0)),
        out_shape=jax.ShapeDtypeStruct((N_NODES, EMB), jnp.float32),
    )(p["eps"].reshape(1), h, a0, a1,
      p["l1"]["w"], p["l1"]["b"], p["l2"]["w"], p["l2"]["b"])


def _pool_body(batch_ref, xr_ref, h_ref, gw1_ref, gb1_ref, gw2_ref, gb2_ref,
               hr_out, henv_out, loss_out,
               hr_acc, s_acc, gs_acc, cnt_acc):
    i = pl.program_id(0)

    @pl.when(i == 0)
    def _():
        hr_acc[...] = jnp.zeros_like(hr_acc)
        s_acc[...] = jnp.zeros_like(s_acc)
        gs_acc[...] = jnp.zeros_like(gs_acc)
        cnt_acc[...] = jnp.zeros_like(cnt_acc)

    b = batch_ref[0, 0, :]
    a = (lax.broadcasted_iota(jnp.int32, (N_GRAPHS, _BR), 0)
         == b[None, :]).astype(jnp.float32)

    xr = xr_ref[...]
    g1 = jnp.dot(xr, gw1_ref[...], preferred_element_type=jnp.float32) + gb1_ref[...]
    g1 = jnp.maximum(g1, 0.0)
    glog = jnp.dot(g1, gw2_ref[...], preferred_element_type=jnp.float32) + gb2_ref[...]
    gate = jax.nn.sigmoid(glog)

    ag = a * gate[:, 0][None, :]
    h = h_ref[...]
    hr_acc[...] += jnp.dot(ag, h, preferred_element_type=jnp.float32)
    s_acc[...] += jnp.dot(a, h, preferred_element_type=jnp.float32)
    gs_acc[...] += jnp.sum(ag, axis=1, keepdims=True)
    cnt_acc[...] += jnp.sum(a, axis=1, keepdims=True)

    @pl.when(i == _NB - 1)
    def _():
        hr = hr_acc[...]
        hr_out[...] = hr
        henv_out[...] = s_acc[...] - hr
        gs = gs_acc[...]
        r_num = gs + 1e-8
        e_num = (cnt_acc[...] - gs) + 1e-8
        ratio = r_num / (r_num + e_num)
        loss_out[...] = jnp.mean(jnp.abs(ratio - GAMMA)).reshape(1, 1)


def _pool(batch3, xr, h_node, gp):
    return pl.pallas_call(
        _pool_body,
        grid=(_NB,),
        in_specs=[
            pl.BlockSpec((1, 1, _BR), lambda i: (i, 0, 0)),
            pl.BlockSpec((_BR, EMB), lambda i: (i, 0)),
            pl.BlockSpec((_BR, EMB), lambda i: (i, 0)),
            pl.BlockSpec((EMB, 2 * EMB), lambda i: (0, 0)),
            pl.BlockSpec((2 * EMB,), lambda i: (0,)),
            pl.BlockSpec((2 * EMB, 1), lambda i: (0, 0)),
            pl.BlockSpec((1,), lambda i: (0,)),
        ],
        out_specs=[
            pl.BlockSpec((N_GRAPHS, EMB), lambda i: (0, 0)),
            pl.BlockSpec((N_GRAPHS, EMB), lambda i: (0, 0)),
            pl.BlockSpec((1, 1), lambda i: (0, 0)),
        ],
        out_shape=[
            jax.ShapeDtypeStruct((N_GRAPHS, EMB), jnp.float32),
            jax.ShapeDtypeStruct((N_GRAPHS, EMB), jnp.float32),
            jax.ShapeDtypeStruct((1, 1), jnp.float32),
        ],
        scratch_shapes=[
            pltpu.VMEM((N_GRAPHS, EMB), jnp.float32),
            pltpu.VMEM((N_GRAPHS, EMB), jnp.float32),
            pltpu.VMEM((N_GRAPHS, 1), jnp.float32),
            pltpu.VMEM((N_GRAPHS, 1), jnp.float32),
        ],
    )(batch3, xr, h_node, gp["l1"]["w"], gp["l1"]["b"],
      gp["l2"]["w"], gp["l2"]["b"])


def _pred_body(hr_ref, henv_ref, pw1_ref, pb1_ref, pw2_ref, pb2_ref,
               rep_out, rem_out):
    i = pl.program_id(0)
    henv = henv_ref[...]
    hr_row = hr_ref[pl.ds(i, 1), :]
    rep = hr_row + ENV_W * henv
    t = jnp.dot(rep, pw1_ref[...], preferred_element_type=jnp.float32) + pb1_ref[...]
    t = jnp.maximum(t, 0.0)
    rep_out[...] = jnp.dot(t, pw2_ref[...], preferred_element_type=jnp.float32) + pb2_ref[...]

    @pl.when(i == 0)
    def _():
        t2 = jnp.dot(hr_ref[...], pw1_ref[...], preferred_element_type=jnp.float32) + pb1_ref[...]
        t2 = jnp.maximum(t2, 0.0)
        rem_out[...] = jnp.dot(t2, pw2_ref[...], preferred_element_type=jnp.float32) + pb2_ref[...]


def _predict(hr, henv, pp):
    return pl.pallas_call(
        _pred_body,
        grid=(N_GRAPHS,),
        in_specs=[
            pl.BlockSpec((N_GRAPHS, EMB), lambda i: (0, 0)),
            pl.BlockSpec((N_GRAPHS, EMB), lambda i: (0, 0)),
            pl.BlockSpec((EMB, 2 * EMB), lambda i: (0, 0)),
            pl.BlockSpec((2 * EMB,), lambda i: (0,)),
            pl.BlockSpec((2 * EMB, NUM_TASKS), lambda i: (0, 0)),
            pl.BlockSpec((NUM_TASKS,), lambda i: (0,)),
        ],
        out_specs=[
            pl.BlockSpec((N_GRAPHS, NUM_TASKS), lambda i: (i, 0)),
            pl.BlockSpec((N_GRAPHS, NUM_TASKS), lambda i: (0, 0)),
        ],
        out_shape=[
            jax.ShapeDtypeStruct((N_GRAPHS * N_GRAPHS, NUM_TASKS), jnp.float32),
            jax.ShapeDtypeStruct((N_GRAPHS, NUM_TASKS), jnp.float32),
        ],
    )(hr, henv, pp["l1"]["w"], pp["l1"]["b"], pp["l2"]["w"], pp["l2"]["b"])


def kernel(x, edge_index, batch, params):
    src, dst = edge_index[0], edge_index[1]
    zeros_block = jnp.zeros((_RPT, EMB), jnp.float32)
    batch3 = batch.reshape(_NB, 1, _BR)

    enc = params["enc"]
    rat = params["rat"]

    a0, a1 = _sc_aggregate(x, src, dst, zeros_block)
    h = _gin_dense(enc[0], x, a0, a1, relu_out=True)
    xr = _gin_dense(rat[0], x, a0, a1, relu_out=True)

    for i in range(1, 5):
        a0, a1 = _sc_aggregate(h, src, dst, zeros_block)
        h = _gin_dense(enc[i], h, a0, a1, relu_out=(i < 4))

    a0, a1 = _sc_aggregate(xr, src, dst, zeros_block)
    xr = _gin_dense(rat[1], xr, a0, a1, relu_out=False)

    hr, henv, loss = _pool(batch3, xr, h, params["gate"])
    pred_rep, pred_rem = _predict(hr, henv, params["pred"])
    return pred_rep, pred_rem, loss.reshape(()), hr
